# baseline (speedup 1.0000x reference)
import functools

import jax
import jax.numpy as jnp
import numpy as np
from jax.experimental import pallas as pl
from jax.experimental.pallas import tpu as pltpu
from jax import shard_map
from jax.sharding import Mesh, PartitionSpec as P

K_OUT = 9
BLK = 2048
ROWS = 128
LANES = 128
BIG = 2**30


def _sweep_kernel(q_ref, kt_ref, qn_ref, kn_ref, pm_ref, v_ref, i_ref,
                  sv_ref, si_ref):
    b = pl.program_id(1)
    nblk = pl.num_programs(1)

    q = q_ref[...]
    kt = kt_ref[...]
    dot = jax.lax.dot_general(
        q, kt, (((1,), (0,)), ((), ())), preferred_element_type=jnp.float32)
    denom = qn_ref[...] * kn_ref[...]
    sim = dot / denom + pm_ref[...]

    @pl.when(b == 0)
    def _init():
        sv_ref[...] = jnp.full(sv_ref.shape, -jnp.inf, jnp.float32)
        si_ref[...] = jnp.zeros(si_ref.shape, jnp.int16)

    svs, sis = [], []
    for j in range(K_OUT):
        svs.append(sv_ref[:, j * LANES:(j + 1) * LANES])
        sis.append(si_ref[:, j * LANES:(j + 1) * LANES])

    nchunk = BLK // LANES
    for c in range(nchunk):
        cv = sim[:, c * LANES:(c + 1) * LANES]
        ci = jnp.full((ROWS, LANES), b * nchunk + c, jnp.int16)
        for j in range(K_OUT):
            rv, ri = svs[j], sis[j]
            up = cv > rv
            svs[j] = jnp.maximum(rv, cv)
            sis[j] = jnp.where(up, ci, ri)
            cv = jnp.minimum(rv, cv)
            ci = jnp.where(up, ri, ci)

    for j in range(K_OUT):
        sv_ref[:, j * LANES:(j + 1) * LANES] = svs[j]
        si_ref[:, j * LANES:(j + 1) * LANES] = sis[j]

    @pl.when(b == nblk - 1)
    def _finish():
        vals = jnp.concatenate(svs, axis=1)
        lane = jax.lax.broadcasted_iota(jnp.int32, vals.shape, 1) % LANES
        gidx = jnp.concatenate(sis, axis=1).astype(jnp.int32) * LANES + lane
        out_v, out_i = [], []
        cur_v = vals
        for _ in range(K_OUT):
            m = jnp.max(cur_v, axis=1, keepdims=True)
            gm = jnp.min(jnp.where(cur_v == m, gidx, BIG), axis=1,
                         keepdims=True)
            out_v.append(m)
            out_i.append(gm)
            kill = (cur_v == m) & (gidx == gm)
            cur_v = jnp.where(kill, -jnp.inf, cur_v)
        v_ref[...] = jnp.concatenate(out_v, axis=1)
        i_ref[...] = jnp.concatenate(out_i, axis=1)


def _merge_kernel(v_ref, i_ref, ov_ref, oi_ref):
    vals = v_ref[...]
    gidx = i_ref[...]
    out_v, out_i = [], []
    cur_v = vals
    for _ in range(K_OUT):
        m = jnp.max(cur_v, axis=1, keepdims=True)
        gm = jnp.min(jnp.where(cur_v == m, gidx, BIG), axis=1, keepdims=True)
        out_v.append(m)
        out_i.append(gm)
        kill = (cur_v == m) & (gidx == gm)
        cur_v = jnp.where(kill, -jnp.inf, cur_v)
    ov_ref[...] = jnp.concatenate(out_v, axis=1)
    oi_ref[...] = jnp.concatenate(out_i, axis=1)


def _local_topk(queries, klocal, pmlocal):
    nq, d = queries.shape
    nkl = klocal.shape[0]
    nblk = (nkl + BLK - 1) // BLK
    npad = nblk * BLK
    nrow = nq // ROWS

    qn = jnp.linalg.norm(queries, axis=1)[:, None]
    kn = jnp.linalg.norm(klocal, axis=1)
    kt = jnp.pad(klocal.T, ((0, 0), (0, npad - nkl)))
    kn = jnp.pad(kn, (0, npad - nkl), constant_values=1.0)[None, :]
    pm = jnp.pad(pmlocal[None, :], ((0, 0), (0, npad - nkl)),
                 constant_values=-jnp.inf)

    return pl.pallas_call(
        _sweep_kernel,
        grid=(nrow, nblk),
        in_specs=[
            pl.BlockSpec((ROWS, d), lambda r, b: (r, 0)),
            pl.BlockSpec((d, BLK), lambda r, b: (0, b)),
            pl.BlockSpec((ROWS, 1), lambda r, b: (r, 0)),
            pl.BlockSpec((1, BLK), lambda r, b: (0, b)),
            pl.BlockSpec((1, BLK), lambda r, b: (0, b)),
        ],
        out_specs=[
            pl.BlockSpec((ROWS, K_OUT), lambda r, b: (r, 0)),
            pl.BlockSpec((ROWS, K_OUT), lambda r, b: (r, 0)),
        ],
        out_shape=[
            jax.ShapeDtypeStruct((nq, K_OUT), jnp.float32),
            jax.ShapeDtypeStruct((nq, K_OUT), jnp.int32),
        ],
        scratch_shapes=[
            pltpu.VMEM((ROWS, K_OUT * LANES), jnp.float32),
            pltpu.VMEM((ROWS, K_OUT * LANES), jnp.int16),
        ],
    )(queries, kt, qn, kn, pm)


def kernel(queries, keys):
    nq, d = queries.shape
    nkeys = keys.shape[0]
    devs = jax.devices()
    ndev = len(devs)

    nglob = ((nkeys + ndev - 1) // ndev) * ndev
    keys_p = jnp.pad(keys, ((0, nglob - nkeys), (0, 0)))
    if nglob > nkeys:
        keys_p = keys_p.at[nkeys:, 0].set(1.0)
    pmask = jnp.pad(jnp.zeros((nkeys,), jnp.float32), (0, nglob - nkeys),
                    constant_values=-jnp.inf)
    nshard = nglob // ndev

    if ndev == 1:
        v, i = _local_topk(queries, keys_p, pmask)
        return v, i

    mesh = Mesh(np.array(devs), ("x",))

    @functools.partial(
        shard_map, mesh=mesh,
        in_specs=(P(), P("x", None), P("x")),
        out_specs=(P(), P()), check_vma=False)
    def _sharded(q, kl, pml):
        shard = jax.lax.axis_index("x")
        lv, li = _local_topk(q, kl, pml)
        li = li + shard * nshard
        av = jax.lax.all_gather(lv, "x")
        ai = jax.lax.all_gather(li, "x")
        av = jnp.moveaxis(av, 0, 1).reshape(nq, ndev * K_OUT)
        ai = jnp.moveaxis(ai, 0, 1).reshape(nq, ndev * K_OUT)
        return pl.pallas_call(
            _merge_kernel,
            out_shape=[
                jax.ShapeDtypeStruct((nq, K_OUT), jnp.float32),
                jax.ShapeDtypeStruct((nq, K_OUT), jnp.int32),
            ],
        )(av, ai)

    v, i = _sharded(queries, keys_p, pmask)
    return v, i


# speedup vs baseline: 1.0757x; 1.0757x over previous
import functools

import jax
import jax.numpy as jnp
import numpy as np
from jax.experimental import pallas as pl
from jax.experimental.pallas import tpu as pltpu
from jax import shard_map
from jax.sharding import Mesh, PartitionSpec as P

K_OUT = 9
BLK = 4096
ROWS = 128
LANES = 128
BIG = 2**30


def _sweep_kernel(q_ref, kt_ref, qn_ref, kn_ref, pm_ref, v_ref, i_ref,
                  sv_ref, si_ref):
    b = pl.program_id(1)
    nblk = pl.num_programs(1)

    q = q_ref[...]
    kt = kt_ref[...]
    dot = jax.lax.dot_general(
        q, kt, (((1,), (0,)), ((), ())), preferred_element_type=jnp.float32)
    denom = qn_ref[...] * kn_ref[...]
    sim = dot / denom + pm_ref[...]

    @pl.when(b == 0)
    def _init():
        sv_ref[...] = jnp.full(sv_ref.shape, -jnp.inf, jnp.float32)
        si_ref[...] = jnp.zeros(si_ref.shape, jnp.int32)

    svs, sis = [], []
    for j in range(K_OUT):
        svs.append(sv_ref[:, j * LANES:(j + 1) * LANES])
        sis.append(si_ref[:, j * LANES:(j + 1) * LANES])

    nchunk = BLK // LANES
    for c in range(nchunk):
        cv = sim[:, c * LANES:(c + 1) * LANES]
        ci = jnp.full((ROWS, LANES), b * nchunk + c, jnp.int32)
        for j in range(K_OUT):
            rv, ri = svs[j], sis[j]
            up = cv > rv
            svs[j] = jnp.maximum(rv, cv)
            sis[j] = jnp.where(up, ci, ri)
            cv = jnp.minimum(rv, cv)
            ci = jnp.where(up, ri, ci)

    for j in range(K_OUT):
        sv_ref[:, j * LANES:(j + 1) * LANES] = svs[j]
        si_ref[:, j * LANES:(j + 1) * LANES] = sis[j]

    @pl.when(b == nblk - 1)
    def _finish():
        vals = jnp.concatenate(svs, axis=1)
        lane = jax.lax.broadcasted_iota(jnp.int32, vals.shape, 1) % LANES
        gidx = jnp.concatenate(sis, axis=1) * LANES + lane
        out_v, out_i = [], []
        cur_v = vals
        for _ in range(K_OUT):
            m = jnp.max(cur_v, axis=1, keepdims=True)
            gm = jnp.min(jnp.where(cur_v == m, gidx, BIG), axis=1,
                         keepdims=True)
            out_v.append(m)
            out_i.append(gm)
            kill = (cur_v == m) & (gidx == gm)
            cur_v = jnp.where(kill, -jnp.inf, cur_v)
        v_ref[...] = jnp.concatenate(out_v, axis=1)
        i_ref[...] = jnp.concatenate(out_i, axis=1)


def _merge_kernel(v_ref, i_ref, ov_ref, oi_ref):
    vals = v_ref[...]
    gidx = i_ref[...]
    out_v, out_i = [], []
    cur_v = vals
    for _ in range(K_OUT):
        m = jnp.max(cur_v, axis=1, keepdims=True)
        gm = jnp.min(jnp.where(cur_v == m, gidx, BIG), axis=1, keepdims=True)
        out_v.append(m)
        out_i.append(gm)
        kill = (cur_v == m) & (gidx == gm)
        cur_v = jnp.where(kill, -jnp.inf, cur_v)
    ov_ref[...] = jnp.concatenate(out_v, axis=1)
    oi_ref[...] = jnp.concatenate(out_i, axis=1)


def _local_topk(queries, klocal, pmlocal):
    nq, d = queries.shape
    nkl = klocal.shape[0]
    nblk = (nkl + BLK - 1) // BLK
    npad = nblk * BLK
    nrow = nq // ROWS

    qn = jnp.linalg.norm(queries, axis=1)[:, None]
    kn = jnp.linalg.norm(klocal, axis=1)
    kt = jnp.pad(klocal.T, ((0, 0), (0, npad - nkl)))
    kn = jnp.pad(kn, (0, npad - nkl), constant_values=1.0)[None, :]
    pm = jnp.pad(pmlocal[None, :], ((0, 0), (0, npad - nkl)),
                 constant_values=-jnp.inf)

    return pl.pallas_call(
        _sweep_kernel,
        grid=(nrow, nblk),
        in_specs=[
            pl.BlockSpec((ROWS, d), lambda r, b: (r, 0)),
            pl.BlockSpec((d, BLK), lambda r, b: (0, b)),
            pl.BlockSpec((ROWS, 1), lambda r, b: (r, 0)),
            pl.BlockSpec((1, BLK), lambda r, b: (0, b)),
            pl.BlockSpec((1, BLK), lambda r, b: (0, b)),
        ],
        out_specs=[
            pl.BlockSpec((ROWS, K_OUT), lambda r, b: (r, 0)),
            pl.BlockSpec((ROWS, K_OUT), lambda r, b: (r, 0)),
        ],
        out_shape=[
            jax.ShapeDtypeStruct((nq, K_OUT), jnp.float32),
            jax.ShapeDtypeStruct((nq, K_OUT), jnp.int32),
        ],
        scratch_shapes=[
            pltpu.VMEM((ROWS, K_OUT * LANES), jnp.float32),
            pltpu.VMEM((ROWS, K_OUT * LANES), jnp.int32),
        ],
    )(queries, kt, qn, kn, pm)


def kernel(queries, keys):
    nq, d = queries.shape
    nkeys = keys.shape[0]
    devs = jax.devices()
    ndev = len(devs)

    nglob = ((nkeys + ndev - 1) // ndev) * ndev
    keys_p = jnp.pad(keys, ((0, nglob - nkeys), (0, 0)))
    if nglob > nkeys:
        keys_p = keys_p.at[nkeys:, 0].set(1.0)
    pmask = jnp.pad(jnp.zeros((nkeys,), jnp.float32), (0, nglob - nkeys),
                    constant_values=-jnp.inf)
    nshard = nglob // ndev

    if ndev == 1:
        v, i = _local_topk(queries, keys_p, pmask)
        return v, i

    mesh = Mesh(np.array(devs), ("x",))

    @functools.partial(
        shard_map, mesh=mesh,
        in_specs=(P(), P("x", None), P("x")),
        out_specs=(P(), P()), check_vma=False)
    def _sharded(q, kl, pml):
        shard = jax.lax.axis_index("x")
        lv, li = _local_topk(q, kl, pml)
        li = li + shard * nshard
        av = jax.lax.all_gather(lv, "x")
        ai = jax.lax.all_gather(li, "x")
        av = jnp.moveaxis(av, 0, 1).reshape(nq, ndev * K_OUT)
        ai = jnp.moveaxis(ai, 0, 1).reshape(nq, ndev * K_OUT)
        return pl.pallas_call(
            _merge_kernel,
            out_shape=[
                jax.ShapeDtypeStruct((nq, K_OUT), jnp.float32),
                jax.ShapeDtypeStruct((nq, K_OUT), jnp.int32),
            ],
        )(av, ai)

    v, i = _sharded(queries, keys_p, pmask)
    return v, i


# speedup vs baseline: 1.2520x; 1.1639x over previous
import functools

import jax
import jax.numpy as jnp
import numpy as np
from jax.experimental import pallas as pl
from jax.experimental.pallas import tpu as pltpu
from jax import shard_map
from jax.sharding import Mesh, PartitionSpec as P

K_OUT = 9
BLK = 4096
ROWS = 128
LANES = 128
BIG = 2**30


def _sweep_kernel(q_ref, kt_ref, qn_ref, kn_ref, pm_ref, v_ref, i_ref,
                  sv_ref, si_ref):
    b = pl.program_id(1)
    nblk = pl.num_programs(1)

    @pl.when(b == 0)
    def _init():
        sv_ref[...] = jnp.full(sv_ref.shape, -jnp.inf, jnp.float32)
        si_ref[...] = jnp.zeros(si_ref.shape, jnp.int32)

    @pl.when(pm_ref[0, 0] == 0.0)
    def _sweep():
        q = q_ref[...]
        kt = kt_ref[...]
        dot = jax.lax.dot_general(
            q, kt, (((1,), (0,)), ((), ())),
            preferred_element_type=jnp.float32)
        denom = qn_ref[...] * kn_ref[...]
        sim = dot / denom + pm_ref[...]

        svs, sis = [], []
        for j in range(K_OUT):
            svs.append(sv_ref[:, j * LANES:(j + 1) * LANES])
            sis.append(si_ref[:, j * LANES:(j + 1) * LANES])

        nchunk = BLK // LANES
        for c in range(nchunk):
            cv = sim[:, c * LANES:(c + 1) * LANES]
            ci = jnp.full((ROWS, LANES), b * nchunk + c, jnp.int32)
            for j in range(K_OUT):
                rv, ri = svs[j], sis[j]
                up = cv > rv
                svs[j] = jnp.maximum(rv, cv)
                sis[j] = jnp.where(up, ci, ri)
                cv = jnp.minimum(rv, cv)
                ci = jnp.where(up, ri, ci)

        for j in range(K_OUT):
            sv_ref[:, j * LANES:(j + 1) * LANES] = svs[j]
            si_ref[:, j * LANES:(j + 1) * LANES] = sis[j]

    @pl.when(b == nblk - 1)
    def _finish():
        vals = sv_ref[...]
        lane = jax.lax.broadcasted_iota(jnp.int32, vals.shape, 1) % LANES
        gidx = si_ref[...] * LANES + lane
        out_v, out_i = [], []
        cur_v = vals
        for _ in range(K_OUT):
            m = jnp.max(cur_v, axis=1, keepdims=True)
            gm = jnp.min(jnp.where(cur_v == m, gidx, BIG), axis=1,
                         keepdims=True)
            out_v.append(m)
            out_i.append(gm)
            kill = (cur_v == m) & (gidx == gm)
            cur_v = jnp.where(kill, -jnp.inf, cur_v)
        v_ref[...] = jnp.concatenate(out_v, axis=1)
        i_ref[...] = jnp.concatenate(out_i, axis=1)


def _merge_kernel(v_ref, i_ref, ov_ref, oi_ref):
    vals = v_ref[...]
    gidx = i_ref[...]
    out_v, out_i = [], []
    cur_v = vals
    for _ in range(K_OUT):
        m = jnp.max(cur_v, axis=1, keepdims=True)
        gm = jnp.min(jnp.where(cur_v == m, gidx, BIG), axis=1, keepdims=True)
        out_v.append(m)
        out_i.append(gm)
        kill = (cur_v == m) & (gidx == gm)
        cur_v = jnp.where(kill, -jnp.inf, cur_v)
    ov_ref[...] = jnp.concatenate(out_v, axis=1)
    oi_ref[...] = jnp.concatenate(out_i, axis=1)


def _local_topk(queries, klocal, pmlocal):
    nq, d = queries.shape
    nkl = klocal.shape[0]
    nblk = (nkl + BLK - 1) // BLK
    npad = nblk * BLK
    nrow = nq // ROWS

    qn = jnp.linalg.norm(queries, axis=1)[:, None]
    kn = jnp.linalg.norm(klocal, axis=1)
    kt = jnp.pad(klocal.T, ((0, 0), (0, npad - nkl)))
    kn = jnp.pad(kn, (0, npad - nkl), constant_values=1.0)[None, :]
    pm = jnp.pad(pmlocal[None, :], ((0, 0), (0, npad - nkl)),
                 constant_values=-jnp.inf)

    return pl.pallas_call(
        _sweep_kernel,
        grid=(nrow, nblk),
        in_specs=[
            pl.BlockSpec((ROWS, d), lambda r, b: (r, 0)),
            pl.BlockSpec((d, BLK), lambda r, b: (0, b)),
            pl.BlockSpec((ROWS, 1), lambda r, b: (r, 0)),
            pl.BlockSpec((1, BLK), lambda r, b: (0, b)),
            pl.BlockSpec((1, BLK), lambda r, b: (0, b)),
        ],
        out_specs=[
            pl.BlockSpec((ROWS, K_OUT), lambda r, b: (r, 0)),
            pl.BlockSpec((ROWS, K_OUT), lambda r, b: (r, 0)),
        ],
        out_shape=[
            jax.ShapeDtypeStruct((nq, K_OUT), jnp.float32),
            jax.ShapeDtypeStruct((nq, K_OUT), jnp.int32),
        ],
        scratch_shapes=[
            pltpu.VMEM((ROWS, K_OUT * LANES), jnp.float32),
            pltpu.VMEM((ROWS, K_OUT * LANES), jnp.int32),
        ],
    )(queries, kt, qn, kn, pm)


def kernel(queries, keys):
    nq, d = queries.shape
    nkeys = keys.shape[0]
    devs = jax.devices()
    ndev = len(devs)

    if ndev == 1:
        v, i = _local_topk(queries, keys, jnp.zeros((nkeys,), jnp.float32))
        return v, i

    if ndev == 2:
        nshard = max((int(0.64 * nkeys) // BLK) * BLK, BLK)
        nshard = max(nshard, nkeys - nshard)
    else:
        nshard = ((nkeys + ndev - 1) // ndev + BLK - 1) // BLK * BLK
    nglob = ndev * nshard
    pad_rows = jnp.zeros((nglob - nkeys, d), keys.dtype).at[:, 0].set(1.0)
    keys_p = jnp.concatenate([keys, pad_rows], axis=0)
    pmask = jnp.pad(jnp.zeros((nkeys,), jnp.float32), (0, nglob - nkeys),
                    constant_values=-jnp.inf)

    mesh = Mesh(np.array(devs), ("x",))

    @functools.partial(
        shard_map, mesh=mesh,
        in_specs=(P(), P("x", None), P("x")),
        out_specs=(P(), P()), check_vma=False)
    def _sharded(q, kl, pml):
        shard = jax.lax.axis_index("x")
        lv, li = _local_topk(q, kl, pml)
        li = li + shard * nshard
        av = jax.lax.all_gather(lv, "x")
        ai = jax.lax.all_gather(li, "x")
        av = jnp.moveaxis(av, 0, 1).reshape(nq, ndev * K_OUT)
        ai = jnp.moveaxis(ai, 0, 1).reshape(nq, ndev * K_OUT)
        return pl.pallas_call(
            _merge_kernel,
            out_shape=[
                jax.ShapeDtypeStruct((nq, K_OUT), jnp.float32),
                jax.ShapeDtypeStruct((nq, K_OUT), jnp.int32),
            ],
        )(av, ai)

    v, i = _sharded(queries, keys_p, pmask)
    return v, i
